# baseline (speedup 1.0000x reference)
import functools

import jax
import jax.numpy as jnp
from jax import lax
from jax.experimental import pallas as pl
from jax.experimental.pallas import tpu as pltpu
from jax.experimental.pallas import tpu_sc as plsc

EPS = 1e-7
LABEL_SMOOTHING = 0.1

NUM_CLASSES = 80
HW = 64 * 64
NC, NS = 2, 16
NW = NC * NS
CHUNK = 128
NCHUNK = HW // CHUNK


def _gather_body(mt_hbm, pred_hbm, out_hbm, mt_v, idx_v, chosen_v, sem):
    wid = lax.axis_index("s") * NC + lax.axis_index("c")

    pltpu.sync_copy(mt_hbm.at[wid], mt_v)

    def _min_step(i, acc):
        return jnp.minimum(acc, mt_v[pl.ds(i * 16, 16)])

    acc = lax.fori_loop(0, HW // 16, _min_step,
                        jnp.full((16,), 2**30, jnp.int32))
    lanes = lax.iota(jnp.int32, 16)
    dnums = lax.GatherDimensionNumbers(offset_dims=(), collapsed_slice_dims=(0,),
                                       start_index_map=(0,))
    for k in (8, 4, 2, 1):
        perm = jnp.bitwise_xor(lanes, k)
        shuf = lax.gather(acc, perm[:, None], dnums, (1,),
                          mode=lax.GatherScatterMode.PROMISE_IN_BOUNDS)
        acc = jnp.minimum(acc, shuf)
    safe_c = jnp.where(acc < NUM_CLASSES, acc, 0)

    base = wid * (HW * NUM_CLASSES) + safe_c
    lane = lanes * NUM_CLASSES

    def _idx_step(k, _):
        row = base + k * (CHUNK * NUM_CLASSES) + lane
        for m in range(CHUNK // 16):
            idx_v[k, pl.ds(m * 16, 16)] = row + m * (16 * NUM_CLASSES)
        return 0

    lax.fori_loop(0, NCHUNK, _idx_step, 0)

    def _fire(k, _):
        pltpu.async_copy(pred_hbm.at[idx_v.at[k]],
                         chosen_v.at[pl.ds(k * CHUNK, CHUNK)], sem)
        return 0

    lax.fori_loop(0, NCHUNK, _fire, 0)
    pltpu.make_async_copy(pred_hbm.at[pl.ds(0, HW)], chosen_v, sem).wait()

    pltpu.sync_copy(chosen_v, out_hbm.at[wid])


def _sc_gather(mask_true_rows, pred_flat):
    return pl.kernel(
        _gather_body,
        out_type=jax.ShapeDtypeStruct((NW, HW), jnp.float32),
        mesh=plsc.VectorSubcoreMesh(core_axis_name="c", subcore_axis_name="s",
                                    num_cores=NC, num_subcores=NS),
        scratch_types=[
            pltpu.VMEM((HW,), jnp.int32),
            pltpu.VMEM((NCHUNK, CHUNK), jnp.int32),
            pltpu.VMEM((HW,), jnp.float32),
            pltpu.SemaphoreType.DMA,
        ],
    )(mask_true_rows, pred_flat)


def _loss_body(mt_ref, ch_ref, out_ref, acc_ref):
    b = pl.program_id(0)
    n = pl.program_id(1)

    @pl.when(n == 0)
    def _init():
        acc_ref[0] = 0.0
        acc_ref[1] = 0.0

    mt = mt_ref[0, 0]
    c = jnp.min(mt)
    valid = c < NUM_CLASSES
    sc = jnp.where(valid, c, 0)

    chosen_pred = ch_ref[0, 0]
    chosen_true = (mt == sc).astype(jnp.float32)
    y = (1.0 - LABEL_SMOOTHING) * chosen_true + LABEL_SMOOTHING / 2.0
    loss = -(y * jnp.log(chosen_pred + EPS)
             + (1.0 - y) * jnp.log(1.0 - chosen_pred + EPS))
    molded = jnp.where(valid, jnp.mean(loss), 0.0)

    acc_ref[0] += molded
    acc_ref[1] += jnp.where(molded != 0.0, 1.0, 0.0)

    @pl.when(n == pl.num_programs(1) - 1)
    def _fin():
        out_ref[b] = acc_ref[0] / (acc_ref[1] + 1.0)


@jax.jit
def kernel(mask_true, mask_pred):
    B, N, H, W = mask_true.shape
    C = mask_pred.shape[-1]

    chosen = _sc_gather(mask_true.reshape(B * N, H * W),
                        mask_pred.reshape(B * N * H * W * C))
    chosen = chosen.reshape(B, N, H, W)

    out = pl.pallas_call(
        _loss_body,
        grid=(B, N),
        in_specs=[
            pl.BlockSpec((1, 1, H, W), lambda b, n: (b, n, 0, 0)),
            pl.BlockSpec((1, 1, H, W), lambda b, n: (b, n, 0, 0)),
        ],
        out_specs=pl.BlockSpec(memory_space=pltpu.SMEM),
        out_shape=jax.ShapeDtypeStruct((B,), jnp.float32),
        scratch_shapes=[pltpu.SMEM((2,), jnp.float32)],
    )(mask_true, chosen)
    return out


# speedup vs baseline: 3.3449x; 3.3449x over previous
import functools

import jax
import jax.numpy as jnp
from jax.experimental import pallas as pl
from jax.experimental.pallas import tpu as pltpu

EPS = 1e-7
LABEL_SMOOTHING = 0.1


def _body(mt_ref, mp_ref, out_ref):
    b = pl.program_id(0)

    mt = mt_ref[0]
    cls = jnp.min(mt, axis=(1, 2), keepdims=True)
    valid = cls < 80
    sc = jnp.where(valid, cls, 0)

    pred = mp_ref[0]
    lane = jax.lax.broadcasted_iota(jnp.int32, pred.shape, 3)
    chosen_pred = jnp.sum(jnp.where(lane == sc[..., None], pred, 0.0), axis=-1)
    chosen_true = (mt == sc).astype(jnp.float32)

    y = (1.0 - LABEL_SMOOTHING) * chosen_true + LABEL_SMOOTHING / 2.0
    loss = -(y * jnp.log(chosen_pred + EPS)
             + (1.0 - y) * jnp.log(1.0 - chosen_pred + EPS))
    molded = jnp.where(valid[:, 0, 0], jnp.mean(loss, axis=(1, 2)), 0.0)
    count = jnp.sum((molded != 0.0).astype(jnp.float32))
    out_ref[b] = jnp.sum(molded) / (count + 1.0)


@jax.jit
def kernel(mask_true, mask_pred):
    B, N, H, W = mask_true.shape
    C = mask_pred.shape[-1]
    out = pl.pallas_call(
        _body,
        grid=(B,),
        in_specs=[
            pl.BlockSpec((1, N, H, W), lambda b: (b, 0, 0, 0)),
            pl.BlockSpec((1, N, H, W, C), lambda b: (b, 0, 0, 0, 0)),
        ],
        out_specs=pl.BlockSpec(memory_space=pltpu.SMEM),
        out_shape=jax.ShapeDtypeStruct((B,), jnp.float32),
    )(mask_true, mask_pred)
    return out


# speedup vs baseline: 3.4014x; 1.0169x over previous
import functools

import jax
import jax.numpy as jnp
from jax.experimental import pallas as pl
from jax.experimental.pallas import tpu as pltpu

EPS = 1e-7
LABEL_SMOOTHING = 0.1

NCH = 4


def _body(mt_ref, mp_ref, out_ref, acc_ref):
    b = pl.program_id(0)
    k = pl.program_id(1)

    @pl.when(k == 0)
    def _init():
        acc_ref[0] = 0.0
        acc_ref[1] = 0.0

    mt = mt_ref[0]
    cls = jnp.min(mt, axis=(1, 2), keepdims=True)
    valid = cls < 80
    sc = jnp.where(valid, cls, 0)

    pred = mp_ref[0]
    lane = jax.lax.broadcasted_iota(jnp.int32, pred.shape, 3)
    chosen_pred = jnp.sum(jnp.where(lane == sc[..., None], pred, 0.0), axis=-1)
    chosen_true = (mt == sc).astype(jnp.float32)

    y = (1.0 - LABEL_SMOOTHING) * chosen_true + LABEL_SMOOTHING / 2.0
    loss = -(y * jnp.log(chosen_pred + EPS)
             + (1.0 - y) * jnp.log(1.0 - chosen_pred + EPS))
    molded = jnp.where(valid[:, 0, 0], jnp.mean(loss, axis=(1, 2)), 0.0)
    acc_ref[0] += jnp.sum(molded)
    acc_ref[1] += jnp.sum((molded != 0.0).astype(jnp.float32))

    @pl.when(k == pl.num_programs(1) - 1)
    def _fin():
        out_ref[b] = acc_ref[0] / (acc_ref[1] + 1.0)


@jax.jit
def kernel(mask_true, mask_pred):
    B, N, H, W = mask_true.shape
    C = mask_pred.shape[-1]
    out = pl.pallas_call(
        _body,
        grid=(B, N // NCH),
        in_specs=[
            pl.BlockSpec((1, NCH, H, W), lambda b, k: (b, k, 0, 0)),
            pl.BlockSpec((1, NCH, H, W, C), lambda b, k: (b, k, 0, 0, 0)),
        ],
        out_specs=pl.BlockSpec(memory_space=pltpu.SMEM),
        out_shape=jax.ShapeDtypeStruct((B,), jnp.float32),
        scratch_shapes=[pltpu.SMEM((2,), jnp.float32)],
    )(mask_true, mask_pred)
    return out
